# baseline (speedup 1.0000x reference)
import functools

import jax
import jax.numpy as jnp
from jax import lax
from jax.experimental import pallas as pl
from jax.experimental.pallas import tpu as pltpu
from jax.experimental.pallas import tpu_sc as plsc

NW = 32
S_OUT = 128
CNT_W = 64
BITS_W = 72
TBLK = 20


def _splat(x, val):
    return jnp.full((16,), val, x)


def _make_counts(P, S_IN, IN_DIM, rpw):
    mesh = plsc.VectorSubcoreMesh(core_axis_name="c", subcore_axis_name="s")

    @functools.partial(
        pl.kernel,
        out_type=jax.ShapeDtypeStruct((P, IN_DIM), jnp.float32),
        mesh=mesh,
        scratch_types=[
            pltpu.VMEM((rpw, S_IN), jnp.int32),
            pltpu.VMEM((rpw, IN_DIM), jnp.float32),
            pltpu.SemaphoreType.DMA,
            pltpu.SemaphoreType.DMA,
        ],
        compiler_params=pltpu.CompilerParams(needs_layout_passes=False),
    )
    def counts_kernel(ids_hbm, c_hbm, ids_v, c_v, sem0, sem1):
        w = lax.axis_index("s") * 2 + lax.axis_index("c")
        base = w * rpw
        pltpu.sync_copy(ids_hbm.at[pl.ds(base, rpw)], ids_v)

        zero16 = jnp.zeros((16,), jnp.float32)
        nchunk = IN_DIM // 16
        ones = jnp.ones((16,), jnp.float32)
        sent = _splat(jnp.int32, IN_DIM)
        sems = (sem0, sem1)
        ngroups = rpw // 8
        assert ngroups * 8 == rpw

        for slot in range(16):
            for k in range(nchunk):
                c_v[slot, pl.ds(16 * k, 16)] = zero16

        pending = {}
        for g in range(ngroups):
            if g >= 2:
                pending.pop(g - 2).wait()
            for r in range(8):
                p = 8 * g + r
                slot = (g % 2) * 8 + r
                srow = _splat(jnp.int32, slot)
                if g >= 2:
                    for v in range(S_IN // 16):
                        idv = ids_v[p - 16, pl.ds(16 * v, 16)]
                        plsc.store_scatter(c_v, [srow, idv], zero16,
                                           mask=idv < sent)
                for v in range(S_IN // 16):
                    idv = ids_v[p, pl.ds(16 * v, 16)]
                    plsc.addupdate_scatter(c_v, [srow, idv], ones,
                                           mask=idv < sent)
            pending[g] = pltpu.async_copy(
                c_v.at[pl.ds((g % 2) * 8, 8)],
                c_hbm.at[pl.ds(base + 8 * g, 8)], sems[g % 2])
        pending.pop(ngroups - 2).wait()
        pending.pop(ngroups - 1).wait()

    return counts_kernel


def _lif_tc(counts, weights, init_state, decay_2d, thresh_2d, seq, batch,
            p_pad):
    units, in_dim = weights.shape
    grid = seq // TBLK

    def body(c_ref, w_ref, init_ref, d_ref, t_ref, states_ref, bits_ref,
             state_scr, syn_scr, sp_scr):
        g = pl.program_id(0)

        @pl.when(g == 0)
        def _():
            state_scr[...] = init_ref[...]

        syn_scr[...] = lax.dot_general(
            c_ref[...], w_ref[...], (((1,), (1,)), ((), ())),
            preferred_element_type=jnp.float32,
            precision=lax.Precision.DEFAULT)

        d = d_ref[...]
        th = t_ref[...]

        st = state_scr[...]
        for t in range(TBLK):
            syn_t = syn_scr[pl.ds(t * batch, batch), :]
            new = st * d + (1.0 - d) * syn_t
            spike = new > th
            st = jnp.where(spike, 0.0, new)
            states_ref[t] = st
            sp_scr[pl.ds(t * batch, batch), :] = spike.astype(jnp.int32)
        state_scr[...] = st

        sp = sp_scr[...]
        words = jnp.zeros((TBLK * batch, 64), jnp.int32)
        for l in range(16):
            words = words + (sp[:, 64 * l:64 * (l + 1)] << l)
        cnt = jnp.sum(sp, axis=1, keepdims=True)
        blk = jnp.concatenate(
            [words, cnt, jnp.zeros((TBLK * batch, BITS_W - 65), jnp.int32)],
            axis=1)
        bits_ref[...] = blk

    return pl.pallas_call(
        body,
        grid=(grid,),
        in_specs=[
            pl.BlockSpec((TBLK * batch, in_dim), lambda i: (i, 0)),
            pl.BlockSpec((units, in_dim), lambda i: (0, 0)),
            pl.BlockSpec((batch, units), lambda i: (0, 0)),
            pl.BlockSpec((1, units), lambda i: (0, 0)),
            pl.BlockSpec((1, units), lambda i: (0, 0)),
        ],
        out_specs=[
            pl.BlockSpec((TBLK, batch, units), lambda i: (i, 0, 0)),
            pl.BlockSpec((TBLK * batch, BITS_W), lambda i: (i, 0)),
        ],
        out_shape=[
            jax.ShapeDtypeStruct((seq, batch, units), jnp.float32),
            jax.ShapeDtypeStruct((p_pad, BITS_W), jnp.int32),
        ],
        scratch_shapes=[
            pltpu.VMEM((batch, units), jnp.float32),
            pltpu.VMEM((TBLK * batch, in_dim), jnp.float32),
            pltpu.VMEM((TBLK * batch, units), jnp.int32),
        ],
    )(counts, weights, init_state, decay_2d, thresh_2d)


def _make_compact(P, P_real, rpw, wpad):
    mesh = plsc.VectorSubcoreMesh(core_axis_name="c", subcore_axis_name="s")
    ngrp = (rpw + 15) // 16

    @functools.partial(
        pl.kernel,
        out_type=(
            jax.ShapeDtypeStruct((P, S_OUT), jnp.int32),
            jax.ShapeDtypeStruct((NW, wpad), jnp.int32),
        ),
        mesh=mesh,
        scratch_types=[
            pltpu.VMEM((rpw, BITS_W), jnp.int32),
            pltpu.VMEM((wpad,), jnp.int32),
            pltpu.VMEM((rpw, S_OUT), jnp.int32),
            pltpu.VMEM((wpad,), jnp.int32),
        ],
        compiler_params=pltpu.CompilerParams(needs_layout_passes=False),
    )
    def compact_kernel(bits_hbm, cnt_hbm, oids_hbm, n_hbm, bits_v, cnt_v,
                       oid_v, n_v):
        w = lax.axis_index("s") * 2 + lax.axis_index("c")
        base = w * rpw
        pltpu.sync_copy(bits_hbm.at[pl.ds(base, rpw)], bits_v)
        pltpu.sync_copy(cnt_hbm.at[w], cnt_v)

        zero16 = jnp.zeros((16,), jnp.int32)
        nchunk = S_OUT // 16

        def zbody(i, _):
            for k in range(nchunk):
                oid_v[i, pl.ds(16 * k, 16)] = zero16
            return 0

        lax.fori_loop(0, rpw, zbody, 0)

        iota = lax.broadcasted_iota(jnp.int32, (16,), 0)

        def row_compact(row, _):
            def lbody(l, cnt):
                for q in range(4):
                    wv = bits_v[row, pl.ds(16 * q, 16)]
                    mb = lax.shift_right_logical(wv, _splat(jnp.int32, l)) & 1
                    m = mb > 0
                    incl = plsc.cumsum(mb)
                    pos = cnt + incl - 1
                    idsv = 64 * l + 16 * q + iota
                    sm = m & (pos < S_OUT)
                    plsc.store_scatter(
                        oid_v, [_splat(jnp.int32, row), pos], idsv, mask=sm)
                    cnt = cnt + jnp.max(incl)
                return cnt

            lax.fori_loop(0, 16, lbody, jnp.int32(0))
            return 0

        def grp(gi, _):
            rows = gi * 16 + iota
            cnts = cnt_v[pl.ds(gi * 16, 16)]
            cnts = jnp.where((rows < rpw) & (base + rows < P_real), cnts, 0)
            n_v[pl.ds(gi * 16, 16)] = jnp.minimum(cnts, S_OUT)
            any_ = jnp.max(cnts)

            @pl.when(any_ > 0)
            def _():
                def dorow(r, _):
                    row = gi * 16 + r
                    c0 = jnp.max(jnp.where(iota == r, cnts, 0))

                    @pl.when(c0 > 0)
                    def _():
                        row_compact(row, None)

                    return 0

                lax.fori_loop(0, 16, dorow, 0)

            return 0

        lax.fori_loop(0, ngrp, grp, 0)
        pltpu.sync_copy(oid_v, oids_hbm.at[pl.ds(base, rpw)])
        pltpu.sync_copy(n_v, n_hbm.at[w])

    return compact_kernel


def kernel(inp_spike_ids, num_inp_spikes, init_state, weights,
           decay_constants, thresholds):
    seq, batch, s_in = inp_spike_ids.shape
    units, in_dim = weights.shape
    P = seq * batch
    rpw = ((P + NW * 8 - 1) // (NW * 8)) * 8
    P_pad = NW * rpw
    wpad = ((((rpw + 15) // 16) * 16 + 127) // 128) * 128

    slot = jnp.arange(s_in, dtype=jnp.int32)[None, None, :]
    ids_sent = jnp.where(slot < num_inp_spikes, inp_spike_ids, in_dim)
    ids2 = jnp.pad(ids_sent.reshape(P, s_in), ((0, P_pad - P), (0, 0)),
                   constant_values=in_dim)

    counts = _make_counts(P_pad, s_in, in_dim, rpw)(ids2)

    states, bits = _lif_tc(
        counts, weights, init_state,
        decay_constants.reshape(1, units), thresholds.reshape(1, units),
        seq, batch, P_pad)

    cnt_wm = jnp.pad(bits[:, CNT_W].reshape(NW, rpw),
                     ((0, 0), (0, wpad - rpw)))
    out_ids2, n_wm = _make_compact(P_pad, P, rpw, wpad)(bits, cnt_wm)

    out_spike_ids = out_ids2[:P].reshape(seq, batch, S_OUT)
    num_out = n_wm[:, :rpw].reshape(P_pad)[:P].reshape(seq, batch, 1)
    return out_spike_ids, num_out, states


# speedup vs baseline: 1.1502x; 1.1502x over previous
import functools

import jax
import jax.numpy as jnp
from jax import lax
from jax.experimental import pallas as pl
from jax.experimental.pallas import tpu as pltpu
from jax.experimental.pallas import tpu_sc as plsc

NW = 32
S_OUT = 128
CNT_W = 64
BITS_W = 72
TBLK = 20


def _splat(x, val):
    return jnp.full((16,), val, x)


def _make_counts(P, S_IN, IN_DIM, rpw):
    mesh = plsc.VectorSubcoreMesh(core_axis_name="c", subcore_axis_name="s")

    @functools.partial(
        pl.kernel,
        out_type=jax.ShapeDtypeStruct((P, IN_DIM), jnp.float32),
        mesh=mesh,
        scratch_types=[
            pltpu.VMEM((rpw, S_IN), jnp.int32),
            pltpu.VMEM((rpw, IN_DIM), jnp.float32),
            pltpu.SemaphoreType.DMA,
            pltpu.SemaphoreType.DMA,
        ],
        compiler_params=pltpu.CompilerParams(needs_layout_passes=False),
    )
    def counts_kernel(ids_hbm, c_hbm, ids_v, c_v, sem0, sem1):
        w = lax.axis_index("s") * 2 + lax.axis_index("c")
        base = w * rpw
        pltpu.sync_copy(ids_hbm.at[pl.ds(base, rpw)], ids_v)

        zero16 = jnp.zeros((16,), jnp.float32)
        nchunk = IN_DIM // 16
        ones = jnp.ones((16,), jnp.float32)
        sent = _splat(jnp.int32, IN_DIM)
        half = ((rpw // 2 + 7) // 8) * 8

        def pair(p, _):
            for k in range(nchunk):
                c_v[p, pl.ds(16 * k, 16)] = zero16
            for v in range(S_IN // 16):
                idv = ids_v[p, pl.ds(16 * v, 16)]
                m = idv < sent
                plsc.addupdate_scatter(
                    c_v, [_splat(jnp.int32, p), idv], ones, mask=m)
            return 0

        lax.fori_loop(0, half, pair, 0)
        cp0 = pltpu.async_copy(
            c_v.at[pl.ds(0, half)], c_hbm.at[pl.ds(base, half)], sem0)
        lax.fori_loop(half, rpw, pair, 0)
        cp1 = pltpu.async_copy(
            c_v.at[pl.ds(half, rpw - half)],
            c_hbm.at[pl.ds(base + half, rpw - half)], sem1)
        cp0.wait()
        cp1.wait()

    return counts_kernel


def _lif_tc(counts, weights, init_state, decay_2d, thresh_2d, seq, batch,
            p_pad):
    units, in_dim = weights.shape
    grid = seq // TBLK

    def body(c_ref, w_ref, init_ref, d_ref, t_ref, states_ref, bits_ref,
             state_scr, syn_scr, sp_scr):
        g = pl.program_id(0)

        @pl.when(g == 0)
        def _():
            state_scr[...] = init_ref[...]

        syn_scr[...] = lax.dot_general(
            c_ref[...], w_ref[...], (((1,), (1,)), ((), ())),
            preferred_element_type=jnp.float32,
            precision=lax.Precision.DEFAULT)

        d = d_ref[...]
        th = t_ref[...]

        st = state_scr[...]
        for t in range(TBLK):
            syn_t = syn_scr[pl.ds(t * batch, batch), :]
            new = st * d + (1.0 - d) * syn_t
            spike = new > th
            st = jnp.where(spike, 0.0, new)
            states_ref[t] = st
            sp_scr[pl.ds(t * batch, batch), :] = spike.astype(jnp.int32)
        state_scr[...] = st

        sp = sp_scr[...]
        words = jnp.zeros((TBLK * batch, 64), jnp.int32)
        for l in range(16):
            words = words + (sp[:, 64 * l:64 * (l + 1)] << l)
        cnt = jnp.sum(sp, axis=1, keepdims=True)
        blk = jnp.concatenate(
            [words, cnt, jnp.zeros((TBLK * batch, BITS_W - 65), jnp.int32)],
            axis=1)
        bits_ref[...] = blk

    return pl.pallas_call(
        body,
        grid=(grid,),
        in_specs=[
            pl.BlockSpec((TBLK * batch, in_dim), lambda i: (i, 0)),
            pl.BlockSpec((units, in_dim), lambda i: (0, 0)),
            pl.BlockSpec((batch, units), lambda i: (0, 0)),
            pl.BlockSpec((1, units), lambda i: (0, 0)),
            pl.BlockSpec((1, units), lambda i: (0, 0)),
        ],
        out_specs=[
            pl.BlockSpec((TBLK, batch, units), lambda i: (i, 0, 0)),
            pl.BlockSpec((TBLK * batch, BITS_W), lambda i: (i, 0)),
        ],
        out_shape=[
            jax.ShapeDtypeStruct((seq, batch, units), jnp.float32),
            jax.ShapeDtypeStruct((p_pad, BITS_W), jnp.int32),
        ],
        scratch_shapes=[
            pltpu.VMEM((batch, units), jnp.float32),
            pltpu.VMEM((TBLK * batch, in_dim), jnp.float32),
            pltpu.VMEM((TBLK * batch, units), jnp.int32),
        ],
    )(counts, weights, init_state, decay_2d, thresh_2d)


def _make_compact(P, P_real, rpw, wpad):
    full_w = (P_real - 1) // rpw
    tail = P_real - full_w * rpw
    mesh = plsc.VectorSubcoreMesh(core_axis_name="c", subcore_axis_name="s")
    ngrp = (rpw + 15) // 16

    @functools.partial(
        pl.kernel,
        out_type=(
            jax.ShapeDtypeStruct((P_real, S_OUT), jnp.int32),
            jax.ShapeDtypeStruct((NW, wpad), jnp.int32),
        ),
        mesh=mesh,
        scratch_types=[
            pltpu.VMEM((rpw, BITS_W), jnp.int32),
            pltpu.VMEM((wpad,), jnp.int32),
            pltpu.VMEM((rpw, S_OUT), jnp.int32),
            pltpu.VMEM((wpad,), jnp.int32),
        ],
        compiler_params=pltpu.CompilerParams(needs_layout_passes=False),
    )
    def compact_kernel(bits_hbm, cnt_hbm, oids_hbm, n_hbm, bits_v, cnt_v,
                       oid_v, n_v):
        w = lax.axis_index("s") * 2 + lax.axis_index("c")
        base = w * rpw
        pltpu.sync_copy(bits_hbm.at[pl.ds(base, rpw)], bits_v)
        pltpu.sync_copy(cnt_hbm.at[w], cnt_v)

        zero16 = jnp.zeros((16,), jnp.int32)
        nchunk = S_OUT // 16

        def zbody(i, _):
            for k in range(nchunk):
                oid_v[i, pl.ds(16 * k, 16)] = zero16
            return 0

        lax.fori_loop(0, rpw, zbody, 0)

        iota = lax.broadcasted_iota(jnp.int32, (16,), 0)

        def row_compact(row, _):
            def lbody(l, cnt):
                for q in range(4):
                    wv = bits_v[row, pl.ds(16 * q, 16)]
                    mb = lax.shift_right_logical(wv, _splat(jnp.int32, l)) & 1
                    m = mb > 0
                    incl = plsc.cumsum(mb)
                    pos = cnt + incl - 1
                    idsv = 64 * l + 16 * q + iota
                    sm = m & (pos < S_OUT)
                    plsc.store_scatter(
                        oid_v, [_splat(jnp.int32, row), pos], idsv, mask=sm)
                    cnt = cnt + jnp.max(incl)
                return cnt

            lax.fori_loop(0, 16, lbody, jnp.int32(0))
            return 0

        def grp(gi, _):
            rows = gi * 16 + iota
            cnts = cnt_v[pl.ds(gi * 16, 16)]
            cnts = jnp.where((rows < rpw) & (base + rows < P_real), cnts, 0)
            n_v[pl.ds(gi * 16, 16)] = jnp.minimum(cnts, S_OUT)
            any_ = jnp.max(cnts)

            @pl.when(any_ > 0)
            def _():
                def dorow(r, _):
                    row = gi * 16 + r
                    c0 = jnp.max(jnp.where(iota == r, cnts, 0))

                    @pl.when(c0 > 0)
                    def _():
                        row_compact(row, None)

                    return 0

                lax.fori_loop(0, 16, dorow, 0)

            return 0

        lax.fori_loop(0, ngrp, grp, 0)

        @pl.when(w < full_w)
        def _():
            pltpu.sync_copy(oid_v, oids_hbm.at[pl.ds(base, rpw)])

        @pl.when(w == full_w)
        def _():
            pltpu.sync_copy(oid_v.at[pl.ds(0, tail)],
                            oids_hbm.at[pl.ds(full_w * rpw, tail)])

        pltpu.sync_copy(n_v, n_hbm.at[w])

    return compact_kernel


def kernel(inp_spike_ids, num_inp_spikes, init_state, weights,
           decay_constants, thresholds):
    seq, batch, s_in = inp_spike_ids.shape
    units, in_dim = weights.shape
    P = seq * batch
    rpw = ((P + NW * 8 - 1) // (NW * 8)) * 8
    P_pad = NW * rpw
    wpad = ((((rpw + 15) // 16) * 16 + 127) // 128) * 128

    slot = jnp.arange(s_in, dtype=jnp.int32)[None, None, :]
    ids_sent = jnp.where(slot < num_inp_spikes, inp_spike_ids, in_dim)
    ids2 = jnp.pad(ids_sent.reshape(P, s_in), ((0, P_pad - P), (0, 0)),
                   constant_values=in_dim)

    counts = _make_counts(P_pad, s_in, in_dim, rpw)(ids2)

    states, bits = _lif_tc(
        counts, weights, init_state,
        decay_constants.reshape(1, units), thresholds.reshape(1, units),
        seq, batch, P_pad)

    cnt_wm = jnp.pad(bits[:, CNT_W].reshape(NW, rpw),
                     ((0, 0), (0, wpad - rpw)))
    out_ids2, n_wm = _make_compact(P_pad, P, rpw, wpad)(bits, cnt_wm)

    out_spike_ids = out_ids2.reshape(seq, batch, S_OUT)
    num_out = n_wm[:, :rpw].reshape(P_pad)[:P].reshape(seq, batch, 1)
    return out_spike_ids, num_out, states


# speedup vs baseline: 1.1554x; 1.0046x over previous
import functools

import jax
import jax.numpy as jnp
from jax import lax
from jax.experimental import pallas as pl
from jax.experimental.pallas import tpu as pltpu
from jax.experimental.pallas import tpu_sc as plsc

NW = 32
S_OUT = 128
CNT_W = 64
BITS_W = 72
TBLK = 25


def _splat(x, val):
    return jnp.full((16,), val, x)


def _make_counts(P, S_IN, IN_DIM, rpw):
    mesh = plsc.VectorSubcoreMesh(core_axis_name="c", subcore_axis_name="s")

    @functools.partial(
        pl.kernel,
        out_type=jax.ShapeDtypeStruct((P, IN_DIM), jnp.float32),
        mesh=mesh,
        scratch_types=[
            pltpu.VMEM((rpw, S_IN), jnp.int32),
            pltpu.VMEM((rpw, IN_DIM), jnp.float32),
            pltpu.SemaphoreType.DMA,
            pltpu.SemaphoreType.DMA,
        ],
        compiler_params=pltpu.CompilerParams(needs_layout_passes=False),
    )
    def counts_kernel(ids_hbm, c_hbm, ids_v, c_v, sem0, sem1):
        w = lax.axis_index("s") * 2 + lax.axis_index("c")
        base = w * rpw
        pltpu.sync_copy(ids_hbm.at[pl.ds(base, rpw)], ids_v)

        zero16 = jnp.zeros((16,), jnp.float32)
        nchunk = IN_DIM // 16
        ones = jnp.ones((16,), jnp.float32)
        sent = _splat(jnp.int32, IN_DIM)
        half = ((rpw // 2 + 7) // 8) * 8

        def pair(p, _):
            for k in range(nchunk):
                c_v[p, pl.ds(16 * k, 16)] = zero16
            for v in range(S_IN // 16):
                idv = ids_v[p, pl.ds(16 * v, 16)]
                m = idv < sent
                plsc.addupdate_scatter(
                    c_v, [_splat(jnp.int32, p), idv], ones, mask=m)
            return 0

        lax.fori_loop(0, half, pair, 0)
        cp0 = pltpu.async_copy(
            c_v.at[pl.ds(0, half)], c_hbm.at[pl.ds(base, half)], sem0)
        lax.fori_loop(half, rpw, pair, 0)
        cp1 = pltpu.async_copy(
            c_v.at[pl.ds(half, rpw - half)],
            c_hbm.at[pl.ds(base + half, rpw - half)], sem1)
        cp0.wait()
        cp1.wait()

    return counts_kernel


def _lif_tc(counts, weights, init_state, decay_2d, thresh_2d, seq, batch,
            p_pad):
    units, in_dim = weights.shape
    grid = seq // TBLK

    def body(c_ref, w_ref, init_ref, d_ref, t_ref, states_ref, bits_ref,
             state_scr, syn_scr, sp_scr):
        g = pl.program_id(0)

        @pl.when(g == 0)
        def _():
            state_scr[...] = init_ref[...]

        syn_scr[...] = lax.dot_general(
            c_ref[...], w_ref[...], (((1,), (1,)), ((), ())),
            preferred_element_type=jnp.float32,
            precision=lax.Precision.DEFAULT)

        d = d_ref[...]
        th = t_ref[...]

        st = state_scr[...]
        for t in range(TBLK):
            syn_t = syn_scr[pl.ds(t * batch, batch), :]
            new = st * d + (1.0 - d) * syn_t
            spike = new > th
            st = jnp.where(spike, 0.0, new)
            states_ref[t] = st
            sp_scr[pl.ds(t * batch, batch), :] = spike.astype(jnp.int32)
        state_scr[...] = st

        sp = sp_scr[...]
        words = jnp.zeros((TBLK * batch, 64), jnp.int32)
        for l in range(16):
            words = words + (sp[:, 64 * l:64 * (l + 1)] << l)
        cnt = jnp.sum(sp, axis=1, keepdims=True)
        blk = jnp.concatenate(
            [words, cnt, jnp.zeros((TBLK * batch, BITS_W - 65), jnp.int32)],
            axis=1)
        bits_ref[...] = blk

    return pl.pallas_call(
        body,
        grid=(grid,),
        in_specs=[
            pl.BlockSpec((TBLK * batch, in_dim), lambda i: (i, 0)),
            pl.BlockSpec((units, in_dim), lambda i: (0, 0)),
            pl.BlockSpec((batch, units), lambda i: (0, 0)),
            pl.BlockSpec((1, units), lambda i: (0, 0)),
            pl.BlockSpec((1, units), lambda i: (0, 0)),
        ],
        out_specs=[
            pl.BlockSpec((TBLK, batch, units), lambda i: (i, 0, 0)),
            pl.BlockSpec((TBLK * batch, BITS_W), lambda i: (i, 0)),
        ],
        out_shape=[
            jax.ShapeDtypeStruct((seq, batch, units), jnp.float32),
            jax.ShapeDtypeStruct((p_pad, BITS_W), jnp.int32),
        ],
        scratch_shapes=[
            pltpu.VMEM((batch, units), jnp.float32),
            pltpu.VMEM((TBLK * batch, in_dim), jnp.float32),
            pltpu.VMEM((TBLK * batch, units), jnp.int32),
        ],
    )(counts, weights, init_state, decay_2d, thresh_2d)


def _make_compact(P, P_real, rpw, wpad):
    full_w = (P_real - 1) // rpw
    tail = P_real - full_w * rpw
    mesh = plsc.VectorSubcoreMesh(core_axis_name="c", subcore_axis_name="s")
    ngrp = (rpw + 15) // 16

    @functools.partial(
        pl.kernel,
        out_type=(
            jax.ShapeDtypeStruct((P_real, S_OUT), jnp.int32),
            jax.ShapeDtypeStruct((NW, wpad), jnp.int32),
        ),
        mesh=mesh,
        scratch_types=[
            pltpu.VMEM((rpw, BITS_W), jnp.int32),
            pltpu.VMEM((wpad,), jnp.int32),
            pltpu.VMEM((rpw, S_OUT), jnp.int32),
            pltpu.VMEM((wpad,), jnp.int32),
        ],
        compiler_params=pltpu.CompilerParams(needs_layout_passes=False),
    )
    def compact_kernel(bits_hbm, cnt_hbm, oids_hbm, n_hbm, bits_v, cnt_v,
                       oid_v, n_v):
        w = lax.axis_index("s") * 2 + lax.axis_index("c")
        base = w * rpw
        pltpu.sync_copy(bits_hbm.at[pl.ds(base, rpw)], bits_v)
        pltpu.sync_copy(cnt_hbm.at[w], cnt_v)

        zero16 = jnp.zeros((16,), jnp.int32)
        nchunk = S_OUT // 16

        def zbody(i, _):
            for k in range(nchunk):
                oid_v[i, pl.ds(16 * k, 16)] = zero16
            return 0

        lax.fori_loop(0, rpw, zbody, 0)

        iota = lax.broadcasted_iota(jnp.int32, (16,), 0)

        def row_compact(row, _):
            def lbody(l, cnt):
                for q in range(4):
                    wv = bits_v[row, pl.ds(16 * q, 16)]
                    mb = lax.shift_right_logical(wv, _splat(jnp.int32, l)) & 1
                    m = mb > 0
                    incl = plsc.cumsum(mb)
                    pos = cnt + incl - 1
                    idsv = 64 * l + 16 * q + iota
                    sm = m & (pos < S_OUT)
                    plsc.store_scatter(
                        oid_v, [_splat(jnp.int32, row), pos], idsv, mask=sm)
                    cnt = cnt + jnp.max(incl)
                return cnt

            lax.fori_loop(0, 16, lbody, jnp.int32(0))
            return 0

        def grp(gi, _):
            rows = gi * 16 + iota
            cnts = cnt_v[pl.ds(gi * 16, 16)]
            cnts = jnp.where((rows < rpw) & (base + rows < P_real), cnts, 0)
            n_v[pl.ds(gi * 16, 16)] = jnp.minimum(cnts, S_OUT)
            any_ = jnp.max(cnts)

            @pl.when(any_ > 0)
            def _():
                def dorow(r, _):
                    row = gi * 16 + r
                    c0 = jnp.max(jnp.where(iota == r, cnts, 0))

                    @pl.when(c0 > 0)
                    def _():
                        row_compact(row, None)

                    return 0

                lax.fori_loop(0, 16, dorow, 0)

            return 0

        lax.fori_loop(0, ngrp, grp, 0)

        @pl.when(w < full_w)
        def _():
            pltpu.sync_copy(oid_v, oids_hbm.at[pl.ds(base, rpw)])

        @pl.when(w == full_w)
        def _():
            pltpu.sync_copy(oid_v.at[pl.ds(0, tail)],
                            oids_hbm.at[pl.ds(full_w * rpw, tail)])

        pltpu.sync_copy(n_v, n_hbm.at[w])

    return compact_kernel


def kernel(inp_spike_ids, num_inp_spikes, init_state, weights,
           decay_constants, thresholds):
    seq, batch, s_in = inp_spike_ids.shape
    units, in_dim = weights.shape
    P = seq * batch
    rpw = ((P + NW * 8 - 1) // (NW * 8)) * 8
    P_pad = NW * rpw
    wpad = ((((rpw + 15) // 16) * 16 + 127) // 128) * 128

    slot = jnp.arange(s_in, dtype=jnp.int32)[None, None, :]
    ids_sent = jnp.where(slot < num_inp_spikes, inp_spike_ids, in_dim)
    ids2 = jnp.pad(ids_sent.reshape(P, s_in), ((0, P_pad - P), (0, 0)),
                   constant_values=in_dim)

    counts = _make_counts(P_pad, s_in, in_dim, rpw)(ids2)

    states, bits = _lif_tc(
        counts, weights, init_state,
        decay_constants.reshape(1, units), thresholds.reshape(1, units),
        seq, batch, P_pad)

    cnt_wm = jnp.pad(bits[:, CNT_W].reshape(NW, rpw),
                     ((0, 0), (0, wpad - rpw)))
    out_ids2, n_wm = _make_compact(P_pad, P, rpw, wpad)(bits, cnt_wm)

    out_spike_ids = out_ids2.reshape(seq, batch, S_OUT)
    num_out = n_wm[:, :rpw].reshape(P_pad)[:P].reshape(seq, batch, 1)
    return out_spike_ids, num_out, states
